# Initial kernel scaffold; baseline (speedup 1.0000x reference)
import functools

import jax
import jax.numpy as jnp
from jax import lax
from jax.experimental import pallas as pl
from jax.experimental.pallas import tpu as pltpu
from jax.experimental.pallas import tpu_sc as plsc

N = 10000
E = 320000
H = 128
T = 3
L = 3

NUM_CORES = 2
NUM_SUBCORES = 16
NUM_WORKERS = NUM_CORES * NUM_SUBCORES
EPW = E // NUM_WORKERS
CHUNK = 128
FULL_CHUNKS = EPW // CHUNK
TAIL = EPW - FULL_CHUNKS * CHUNK
PAIRS = FULL_CHUNKS // 2
ROWS_PER_TILE = N // NUM_SUBCORES
FLUSH_CHUNK = 125


def _make_sc_agg(num_towers: int):
    mesh = plsc.VectorSubcoreMesh(core_axis_name="c", subcore_axis_name="s")

    def body(h_hbm, src_hbm, dst_hbm, zeros_hbm, out_hbm,
             idx0, idx1, dst0, dst1, idxT, dstT,
             rows0, rows1, rowsT, zbuf, acc, sem0, sem1):
        c = lax.axis_index("c")
        s = lax.axis_index("s")
        wid = c * NUM_SUBCORES + s
        base = wid * EPW

        pltpu.sync_copy(zeros_hbm, zbuf)

        for t in range(num_towers):
            r0 = s * ROWS_PER_TILE
            for k in range(ROWS_PER_TILE // FLUSH_CHUNK):
                pltpu.sync_copy(
                    zbuf.at[pl.ds(0, FLUSH_CHUNK)],
                    acc.at[pl.ds(r0 + k * FLUSH_CHUNK, FLUSH_CHUNK)])
            plsc.subcore_barrier()

            table = h_hbm.at[t]

            def fire(j, idxbuf, rowsbuf, sem):
                pltpu.sync_copy(src_hbm.at[pl.ds(base + j * CHUNK, CHUNK)],
                                idxbuf)
                pltpu.async_copy(table.at[idxbuf], rowsbuf, sem)

            def wait_rows(idxbuf, rowsbuf, sem):
                pltpu.make_async_copy(table.at[idxbuf], rowsbuf, sem).wait()

            def scat(j, dstbuf, rowsbuf):
                pltpu.sync_copy(dst_hbm.at[pl.ds(base + j * CHUNK, CHUNK)],
                                dstbuf)
                pltpu.sync_copy(rowsbuf, acc.at[dstbuf], add=True)

            fire(0, idx0, rows0, sem0)
            fire(1, idx1, rows1, sem1)

            def pair_body(p, carry):
                j0 = 2 * p
                wait_rows(idx0, rows0, sem0)
                scat(j0, dst0, rows0)
                fire(j0 + 2, idx0, rows0, sem0)
                wait_rows(idx1, rows1, sem1)
                scat(j0 + 1, dst1, rows1)
                fire(j0 + 3, idx1, rows1, sem1)
                return carry

            lax.fori_loop(0, PAIRS - 1, pair_body, 0)

            j_last = 2 * (PAIRS - 1)
            wait_rows(idx0, rows0, sem0)
            scat(j_last, dst0, rows0)
            wait_rows(idx1, rows1, sem1)
            scat(j_last + 1, dst1, rows1)

            tb = base + FULL_CHUNKS * CHUNK
            pltpu.sync_copy(src_hbm.at[pl.ds(tb, TAIL)], idxT)
            pltpu.async_copy(table.at[idxT], rowsT, sem0).wait()
            pltpu.sync_copy(dst_hbm.at[pl.ds(tb, TAIL)], dstT)
            pltpu.sync_copy(rowsT, acc.at[dstT], add=True)

            plsc.subcore_barrier()

            for k in range(ROWS_PER_TILE // FLUSH_CHUNK):
                off = r0 + k * FLUSH_CHUNK
                pltpu.sync_copy(acc.at[pl.ds(off, FLUSH_CHUNK)],
                                rows0.at[pl.ds(0, FLUSH_CHUNK)])
                pltpu.sync_copy(rows0.at[pl.ds(0, FLUSH_CHUNK)],
                                out_hbm.at[c, t, pl.ds(off, FLUSH_CHUNK)])
            plsc.subcore_barrier()

    return pl.kernel(
        body,
        out_type=jax.ShapeDtypeStruct((NUM_CORES, num_towers, N, H),
                                      jnp.float32),
        mesh=mesh,
        scratch_types=[
            pltpu.VMEM((CHUNK,), jnp.int32),
            pltpu.VMEM((CHUNK,), jnp.int32),
            pltpu.VMEM((CHUNK,), jnp.int32),
            pltpu.VMEM((CHUNK,), jnp.int32),
            pltpu.VMEM((TAIL,), jnp.int32),
            pltpu.VMEM((TAIL,), jnp.int32),
            pltpu.VMEM((CHUNK, H), jnp.float32),
            pltpu.VMEM((CHUNK, H), jnp.float32),
            pltpu.VMEM((TAIL, H), jnp.float32),
            pltpu.VMEM((CHUNK, H), jnp.float32),
            pltpu.VMEM_SHARED((N, H), jnp.float32),
            pltpu.SemaphoreType.DMA,
            pltpu.SemaphoreType.DMA,
        ],
    )


_sc_agg_1 = _make_sc_agg(1)
_sc_agg_3 = _make_sc_agg(T)

BN = 1000
GRID = N // BN


def _mm(a, w):
    return lax.dot_general(a, w, (((1,), (0,)), ((), ())),
                           preferred_element_type=jnp.float32,
                           precision=lax.Precision.HIGHEST)


def _mlp_first_body(scale_ref, x_ref, aggp_ref, w1_ref, b1_ref, w2_ref,
                    b2_ref, out_ref):
    agg = aggp_ref[0] + aggp_ref[1]
    x = x_ref[...]
    for t in range(T):
        u = scale_ref[t] * x + agg
        v = jnp.maximum(_mm(u, w1_ref[t]) + b1_ref[t], 0.0)
        w = jnp.maximum(_mm(v, w2_ref[t]) + b2_ref[t], 0.0)
        out_ref[t] = w


def _mlp_mid_body(scale_ref, h_ref, aggp_ref, w1_ref, b1_ref, w2_ref,
                  b2_ref, out_ref):
    for t in range(T):
        u = scale_ref[t] * h_ref[t] + (aggp_ref[0, t] + aggp_ref[1, t])
        v = jnp.maximum(_mm(u, w1_ref[t]) + b1_ref[t], 0.0)
        w = jnp.maximum(_mm(v, w2_ref[t]) + b2_ref[t], 0.0)
        out_ref[t] = w


_W_SPEC = pl.BlockSpec((T, H, H), lambda i: (0, 0, 0))
_B_SPEC = pl.BlockSpec((T, H), lambda i: (0, 0))
_H3_SPEC = pl.BlockSpec((T, BN, H), lambda i: (0, i, 0))

_mlp_first = pl.pallas_call(
    _mlp_first_body,
    grid=(GRID,),
    in_specs=[
        pl.BlockSpec(memory_space=pltpu.SMEM),
        pl.BlockSpec((BN, H), lambda i: (i, 0)),
        pl.BlockSpec((NUM_CORES, BN, H), lambda i: (0, i, 0)),
        _W_SPEC, _B_SPEC, _W_SPEC, _B_SPEC,
    ],
    out_specs=_H3_SPEC,
    out_shape=jax.ShapeDtypeStruct((T, N, H), jnp.float32),
)

_mlp_mid = pl.pallas_call(
    _mlp_mid_body,
    grid=(GRID,),
    in_specs=[
        pl.BlockSpec(memory_space=pltpu.SMEM),
        _H3_SPEC,
        pl.BlockSpec((NUM_CORES, T, BN, H), lambda i: (0, 0, i, 0)),
        _W_SPEC, _B_SPEC, _W_SPEC, _B_SPEC,
    ],
    out_specs=_H3_SPEC,
    out_shape=jax.ShapeDtypeStruct((T, N, H), jnp.float32),
)


def kernel(x, edge_index, W1, b1, W2, b2, eps):
    src = edge_index[0]
    dst = edge_index[1]
    scale = 1.0 + eps
    zeros = jnp.zeros((CHUNK, H), jnp.float32)

    aggp0 = _sc_agg_1(x[None], src, dst, zeros)
    h = _mlp_first(scale[:, 0], x, aggp0[:, 0],
                   W1[:, 0], b1[:, 0], W2[:, 0], b2[:, 0])
    for l in range(1, L):
        aggp = _sc_agg_3(h, src, dst, zeros)
        h = _mlp_mid(scale[:, l], h, aggp,
                     W1[:, l], b1[:, l], W2[:, l], b2[:, l])
    return jnp.transpose(h, (1, 0, 2))


# speedup vs baseline: 7.4571x; 7.4571x over previous
import functools

import jax
import jax.numpy as jnp
from jax import lax
from jax.experimental import pallas as pl
from jax.experimental.pallas import tpu as pltpu
from jax.experimental.pallas import tpu_sc as plsc

N = 10000
NP = 10240
E = 320000
H = 128
T = 3
L = 3

NUM_CORES = 2
NUM_SUBCORES = 16
NUM_WORKERS = NUM_CORES * NUM_SUBCORES
EPW = E // NUM_WORKERS
CHUNK = 128
FULL_CHUNKS = EPW // CHUNK
TAIL = EPW - FULL_CHUNKS * CHUNK
PAIRS = FULL_CHUNKS // 2
ROWS_PER_TILE = NP // NUM_SUBCORES
FLUSH_CHUNK = 128


def _make_sc_agg(num_towers: int):
    mesh = plsc.VectorSubcoreMesh(core_axis_name="c", subcore_axis_name="s")

    def body(h_hbm, src_hbm, dst_hbm, zeros_hbm, out_hbm,
             idx0, idx1, dst0, dst1, idxT, dstT,
             rows0, rows1, rowsT, acc, sem0, sem1):
        c = lax.axis_index("c")
        s = lax.axis_index("s")
        wid = c * NUM_SUBCORES + s
        base = wid * EPW

        for t in range(num_towers):
            pltpu.sync_copy(zeros_hbm, rows1)
            r0 = s * ROWS_PER_TILE
            for k in range(ROWS_PER_TILE // FLUSH_CHUNK):
                pltpu.sync_copy(
                    rows1.at[pl.ds(0, FLUSH_CHUNK)],
                    acc.at[pl.ds(r0 + k * FLUSH_CHUNK, FLUSH_CHUNK)])
            plsc.subcore_barrier()

            table = h_hbm.at[t]

            def fire(j, idxbuf, rowsbuf, sem):
                pltpu.sync_copy(src_hbm.at[pl.ds(base + j * CHUNK, CHUNK)],
                                idxbuf)
                pltpu.async_copy(table.at[idxbuf], rowsbuf, sem)

            def wait_rows(idxbuf, rowsbuf, sem):
                pltpu.make_async_copy(table.at[idxbuf], rowsbuf, sem).wait()

            def scat(j, dstbuf, rowsbuf):
                pltpu.sync_copy(dst_hbm.at[pl.ds(base + j * CHUNK, CHUNK)],
                                dstbuf)
                pltpu.sync_copy(rowsbuf, acc.at[dstbuf], add=True)

            fire(0, idx0, rows0, sem0)
            fire(1, idx1, rows1, sem1)

            def pair_body(p, carry):
                j0 = 2 * p
                wait_rows(idx0, rows0, sem0)
                scat(j0, dst0, rows0)
                fire(j0 + 2, idx0, rows0, sem0)
                wait_rows(idx1, rows1, sem1)
                scat(j0 + 1, dst1, rows1)
                fire(j0 + 3, idx1, rows1, sem1)
                return carry

            lax.fori_loop(0, PAIRS - 1, pair_body, 0)

            j_last = 2 * (PAIRS - 1)
            wait_rows(idx0, rows0, sem0)
            scat(j_last, dst0, rows0)
            wait_rows(idx1, rows1, sem1)
            scat(j_last + 1, dst1, rows1)

            tb = base + FULL_CHUNKS * CHUNK
            pltpu.sync_copy(src_hbm.at[pl.ds(tb, TAIL)], idxT)
            pltpu.async_copy(table.at[idxT], rowsT, sem0).wait()
            pltpu.sync_copy(dst_hbm.at[pl.ds(tb, TAIL)], dstT)
            pltpu.sync_copy(rowsT, acc.at[dstT], add=True)

            plsc.subcore_barrier()

            for k in range(ROWS_PER_TILE // FLUSH_CHUNK):
                off = r0 + k * FLUSH_CHUNK
                pltpu.sync_copy(acc.at[pl.ds(off, FLUSH_CHUNK)],
                                rows0.at[pl.ds(0, FLUSH_CHUNK)])
                pltpu.sync_copy(rows0.at[pl.ds(0, FLUSH_CHUNK)],
                                out_hbm.at[c, t, pl.ds(off, FLUSH_CHUNK)])
            plsc.subcore_barrier()

    return pl.kernel(
        body,
        out_type=jax.ShapeDtypeStruct((NUM_CORES, num_towers, NP, H),
                                      jnp.float32),
        mesh=mesh,
        scratch_types=[
            pltpu.VMEM((CHUNK,), jnp.int32),
            pltpu.VMEM((CHUNK,), jnp.int32),
            pltpu.VMEM((CHUNK,), jnp.int32),
            pltpu.VMEM((CHUNK,), jnp.int32),
            pltpu.VMEM((TAIL,), jnp.int32),
            pltpu.VMEM((TAIL,), jnp.int32),
            pltpu.VMEM((CHUNK, H), jnp.float32),
            pltpu.VMEM((CHUNK, H), jnp.float32),
            pltpu.VMEM((TAIL, H), jnp.float32),
            pltpu.VMEM_SHARED((NP, H), jnp.float32),
            pltpu.SemaphoreType.DMA,
            pltpu.SemaphoreType.DMA,
        ],
    )


_sc_agg_1 = _make_sc_agg(1)
_sc_agg_3 = _make_sc_agg(T)

BN = 1024
GRID = NP // BN


def _mm(a, w):
    return lax.dot_general(a, w, (((1,), (0,)), ((), ())),
                           preferred_element_type=jnp.float32,
                           precision=lax.Precision.HIGHEST)


def _mlp_first_body(scale_ref, x_ref, aggp_ref, w1_ref, b1_ref, w2_ref,
                    b2_ref, out_ref):
    agg = aggp_ref[0] + aggp_ref[1]
    x = x_ref[...]
    for t in range(T):
        u = scale_ref[t] * x + agg
        v = jnp.maximum(_mm(u, w1_ref[t]) + b1_ref[t], 0.0)
        w = jnp.maximum(_mm(v, w2_ref[t]) + b2_ref[t], 0.0)
        out_ref[t] = w


def _mlp_mid_body(scale_ref, h_ref, aggp_ref, w1_ref, b1_ref, w2_ref,
                  b2_ref, out_ref):
    for t in range(T):
        u = scale_ref[t] * h_ref[t] + (aggp_ref[0, t] + aggp_ref[1, t])
        v = jnp.maximum(_mm(u, w1_ref[t]) + b1_ref[t], 0.0)
        w = jnp.maximum(_mm(v, w2_ref[t]) + b2_ref[t], 0.0)
        out_ref[t] = w


_W_SPEC = pl.BlockSpec((T, H, H), lambda i: (0, 0, 0))
_B_SPEC = pl.BlockSpec((T, H), lambda i: (0, 0))
_H3_SPEC = pl.BlockSpec((T, BN, H), lambda i: (0, i, 0))

_mlp_first = pl.pallas_call(
    _mlp_first_body,
    grid=(GRID,),
    in_specs=[
        pl.BlockSpec(memory_space=pltpu.SMEM),
        pl.BlockSpec((BN, H), lambda i: (i, 0)),
        pl.BlockSpec((NUM_CORES, BN, H), lambda i: (0, i, 0)),
        _W_SPEC, _B_SPEC, _W_SPEC, _B_SPEC,
    ],
    out_specs=_H3_SPEC,
    out_shape=jax.ShapeDtypeStruct((T, NP, H), jnp.float32),
)

_mlp_mid = pl.pallas_call(
    _mlp_mid_body,
    grid=(GRID,),
    in_specs=[
        pl.BlockSpec(memory_space=pltpu.SMEM),
        _H3_SPEC,
        pl.BlockSpec((NUM_CORES, T, BN, H), lambda i: (0, 0, i, 0)),
        _W_SPEC, _B_SPEC, _W_SPEC, _B_SPEC,
    ],
    out_specs=_H3_SPEC,
    out_shape=jax.ShapeDtypeStruct((T, NP, H), jnp.float32),
)


def kernel(x, edge_index, W1, b1, W2, b2, eps):
    src = edge_index[0]
    dst = edge_index[1]
    scale = 1.0 + eps
    zeros = jnp.zeros((CHUNK, H), jnp.float32)
    xp = jnp.pad(x, ((0, NP - N), (0, 0)))

    aggp0 = _sc_agg_1(xp[None], src, dst, zeros)
    h = _mlp_first(scale[:, 0], xp, aggp0[:, 0],
                   W1[:, 0], b1[:, 0], W2[:, 0], b2[:, 0])
    for l in range(1, L):
        aggp = _sc_agg_3(h, src, dst, zeros)
        h = _mlp_mid(scale[:, l], h, aggp,
                     W1[:, l], b1[:, l], W2[:, l], b2[:, l])
    return jnp.transpose(h[:, :N], (1, 0, 2))
